# baseline (speedup 1.0000x reference)
import functools

import jax
import jax.numpy as jnp
from jax import lax
from jax.experimental import pallas as pl
from jax.experimental.pallas import tpu as pltpu
from jax.experimental.pallas import tpu_sc as plsc

BATCH = 16384
EMBED_K = 64
L = 16
NC = 2
NS = 16
NW = NC * NS
BPW = BATCH // NW
CHUNK = 128
NCH = BPW // CHUNK


def _lane_consts():
    lanes = lax.iota(jnp.int32, L)
    return (lanes % (L // 2)) * 2, lanes < (L // 2)


def _perm(a, idx):
    return lax.gather(
        a, idx[:, None],
        dimension_numbers=lax.GatherDimensionNumbers(
            offset_dims=(), collapsed_slice_dims=(0,),
            start_index_map=(0,)),
        slice_sizes=(1,),
        mode=lax.GatherScatterMode.PROMISE_IN_BOUNDS)


def _u_body(x_hbm, w_hbm, stage_hbm, uidx_v, u_rows, sem):
    wid = lax.axis_index("s") * NC + lax.axis_index("c")
    base = wid * BPW

    pltpu.sync_copy(x_hbm.at[0, pl.ds(base, BPW)], uidx_v)

    def row_body(g, _):
        ivu = uidx_v[pl.ds(g * L, L)]
        for k in range(L):
            r = g * L + k
            pltpu.async_copy(w_hbm.at[ivu[k]], u_rows.at[r], sem)
        return _

    lax.fori_loop(0, BPW // L, row_body, 0, unroll=False)
    pltpu.make_async_copy(w_hbm.at[pl.ds(0, BPW)], u_rows, sem).wait()
    pltpu.sync_copy(u_rows, stage_hbm.at[pl.ds(base, BPW)])


def _v_body(x_hbm, h_hbm, stage_hbm, out_hbm,
            vidx_v, us_v, v_rows, out_v, sems):
    wid = lax.axis_index("s") * NC + lax.axis_index("c")
    base = wid * BPW
    idx_even, lo_mask = _lane_consts()
    idx_odd = idx_even + 1

    pltpu.sync_copy(x_hbm.at[1, pl.ds(base, BPW)], vidx_v)
    pltpu.sync_copy(stage_hbm.at[pl.ds(base, BPW)], us_v)

    def fire(j):
        buf = j % 2

        def row_body(g, _):
            ivv = vidx_v[pl.ds(j * CHUNK + g * L, L)]
            for k in range(L):
                r = g * L + k
                pltpu.async_copy(h_hbm.at[ivv[k]],
                                 v_rows.at[buf, r], sems.at[buf])
            return _

        lax.fori_loop(0, CHUNK // L, row_body, 0, unroll=False)

    def drain(j):
        buf = j % 2
        pltpu.make_async_copy(h_hbm.at[pl.ds(0, CHUNK)],
                              v_rows.at[buf], sems.at[buf]).wait()

    def _hadd(a, b):
        ce = jnp.where(lo_mask, _perm(a, idx_even), _perm(b, idx_even))
        co = jnp.where(lo_mask, _perm(a, idx_odd), _perm(b, idx_odd))
        return ce + co

    fire(0)
    fire(1)

    for j in range(NCH):
        buf = j % 2
        drain(j)

        def group_body(g, _, buf=buf, cbase=j * CHUNK):
            vecs = []
            for k in range(L):
                r = g * L + k
                ub = cbase + r
                acc = (us_v[ub, pl.ds(0, L)] *
                       v_rows[buf, r, pl.ds(0, L)])
                for m in range(1, EMBED_K // L):
                    acc = acc + (us_v[ub, pl.ds(m * L, L)] *
                                 v_rows[buf, r, pl.ds(m * L, L)])
                vecs.append(acc)
            while len(vecs) > 1:
                vecs = [_hadd(vecs[i], vecs[i + 1])
                        for i in range(0, len(vecs), 2)]
            sums = vecs[0]
            out_v[pl.ds(cbase + g * L, L)] = 1.0 / (1.0 + jnp.exp(-sums))
            return _

        lax.fori_loop(0, CHUNK // L, group_body, 0, unroll=False)
        if j + 2 < NCH:
            fire(j + 2)

    pltpu.sync_copy(out_v, out_hbm.at[pl.ds(base, BPW)])


@jax.jit
def kernel(x, W, H):
    mesh = plsc.VectorSubcoreMesh(core_axis_name="c", subcore_axis_name="s")
    f1 = functools.partial(
        pl.kernel, mesh=mesh,
        compiler_params=pltpu.CompilerParams(use_tc_tiling_on_sc=True),
        out_type=jax.ShapeDtypeStruct((BATCH, EMBED_K), jnp.float32),
        scratch_types=[
            pltpu.VMEM((BPW,), jnp.int32),
            pltpu.VMEM((BPW, EMBED_K), jnp.float32),
            pltpu.SemaphoreType.DMA,
        ],
    )(_u_body)
    f2 = functools.partial(
        pl.kernel, mesh=mesh,
        compiler_params=pltpu.CompilerParams(use_tc_tiling_on_sc=True),
        out_type=jax.ShapeDtypeStruct((BATCH,), jnp.float32),
        scratch_types=[
            pltpu.VMEM((BPW,), jnp.int32),
            pltpu.VMEM((BPW, EMBED_K), jnp.float32),
            pltpu.VMEM((2, CHUNK, EMBED_K), jnp.float32),
            pltpu.VMEM((BPW,), jnp.float32),
            pltpu.SemaphoreType.DMA((2,)),
        ],
    )(_v_body)
    x_t = x.astype(jnp.int32).T
    stage = f1(x_t, W)
    return f2(x_t, H, stage)


# speedup vs baseline: 1.0406x; 1.0406x over previous
import functools

import jax
import jax.numpy as jnp
from jax import lax
from jax.experimental import pallas as pl
from jax.experimental.pallas import tpu as pltpu
from jax.experimental.pallas import tpu_sc as plsc

BATCH = 16384
EMBED_K = 64
L = 16
NC = 2
NS = 16
NW = NC * NS
BPW = BATCH // NW
CHUNK = 128
NCH = BPW // CHUNK


def _sc_body(x_hbm, w_hbm, h_hbm, out_hbm,
             uidx_v, vidx_v, u_rows, v_rows, out_v, sems):
    wid = lax.axis_index("s") * NC + lax.axis_index("c")
    base = wid * BPW

    lane_ids = lax.iota(jnp.int32, L)
    idx_even = (lane_ids % (L // 2)) * 2
    idx_odd = idx_even + 1
    lo_mask = lane_ids < (L // 2)

    def _perm(a, idx):
        return lax.gather(
            a, idx[:, None],
            dimension_numbers=lax.GatherDimensionNumbers(
                offset_dims=(), collapsed_slice_dims=(0,),
                start_index_map=(0,)),
            slice_sizes=(1,),
            mode=lax.GatherScatterMode.PROMISE_IN_BOUNDS)

    pltpu.sync_copy(x_hbm.at[0, pl.ds(base, BPW)], uidx_v)
    pltpu.sync_copy(x_hbm.at[1, pl.ds(base, BPW)], vidx_v)

    def fire(j):
        buf = j % 2

        def row_body(g, _):
            ivu = uidx_v[pl.ds(j * CHUNK + g * L, L)]
            ivv = vidx_v[pl.ds(j * CHUNK + g * L, L)]
            for k in range(L):
                r = g * L + k
                pltpu.async_copy(w_hbm.at[ivu[k]],
                                 u_rows.at[buf, r], sems.at[buf])
                pltpu.async_copy(h_hbm.at[ivv[k]],
                                 v_rows.at[buf, r], sems.at[buf])
            return _

        lax.fori_loop(0, CHUNK // L, row_body, 0, unroll=False)

    def drain(j):
        buf = j % 2
        pltpu.make_async_copy(w_hbm.at[pl.ds(0, CHUNK)],
                              u_rows.at[buf], sems.at[buf]).wait()
        pltpu.make_async_copy(h_hbm.at[pl.ds(0, CHUNK)],
                              v_rows.at[buf], sems.at[buf]).wait()

    def _hadd(a, b):
        ce = jnp.where(lo_mask, _perm(a, idx_even), _perm(b, idx_even))
        co = jnp.where(lo_mask, _perm(a, idx_odd), _perm(b, idx_odd))
        return ce + co

    fire(0)
    fire(1)

    for j in range(NCH):
        buf = j % 2
        drain(j)

        def group_body(g, _, buf=buf, cbase=j * CHUNK):
            vecs = []
            for k in range(L):
                r = g * L + k
                acc = (u_rows[buf, r, pl.ds(0, L)] *
                       v_rows[buf, r, pl.ds(0, L)])
                for m in range(1, EMBED_K // L):
                    acc = acc + (u_rows[buf, r, pl.ds(m * L, L)] *
                                 v_rows[buf, r, pl.ds(m * L, L)])
                vecs.append(acc)
            while len(vecs) > 1:
                vecs = [_hadd(vecs[i], vecs[i + 1])
                        for i in range(0, len(vecs), 2)]
            sums = vecs[0]
            out_v[pl.ds(cbase + g * L, L)] = 1.0 / (1.0 + jnp.exp(-sums))
            return _

        lax.fori_loop(0, CHUNK // L, group_body, 0, unroll=False)
        if j + 2 < NCH:
            fire(j + 2)

    pltpu.sync_copy(out_v, out_hbm.at[pl.ds(base, BPW)])


@jax.jit
def kernel(x, W, H):
    mesh = plsc.VectorSubcoreMesh(core_axis_name="c", subcore_axis_name="s")
    f = functools.partial(
        pl.kernel, mesh=mesh,
        compiler_params=pltpu.CompilerParams(use_tc_tiling_on_sc=True),
        out_type=jax.ShapeDtypeStruct((BATCH,), jnp.float32),
        scratch_types=[
            pltpu.VMEM((BPW,), jnp.int32),
            pltpu.VMEM((BPW,), jnp.int32),
            pltpu.VMEM((2, CHUNK, EMBED_K), jnp.float32),
            pltpu.VMEM((2, CHUNK, EMBED_K), jnp.float32),
            pltpu.VMEM((BPW,), jnp.float32),
            pltpu.SemaphoreType.DMA((2,)),
        ],
    )(_sc_body)
    return f(x.astype(jnp.int32).T, W, H)
